# baseline (speedup 1.0000x reference)
import functools
import math

import jax
import jax.numpy as jnp
from jax.experimental import pallas as pl
from jax.experimental.pallas import tpu as pltpu

_NH = 12


def _fused_kernel(x_ref, wqkv_ref, wo_ref, bqkv_ref, bo_ref, g_ref,
                  be_ref, o_ref, *, nb, sb, dh, eps):
    x = x_ref[...]
    xb = x.astype(jnp.bfloat16)
    dn = (((1,), (1,)), ((), ()))
    H = x.shape[1]

    qkv = jax.lax.dot_general(xb, wqkv_ref[...], dn,
                              preferred_element_type=jnp.float32)
    qkvb = (qkv + bqkv_ref[...]).astype(jnp.bfloat16)
    qb = qkvb[:, :H]
    kb = qkvb[:, H:2 * H]
    vb = qkvb[:, 2 * H:]

    ones_dh = jnp.ones((sb, dh), dtype=jnp.bfloat16)
    pairs = [(slice(b * sb, (b + 1) * sb), slice(h * dh, (h + 1) * dh))
             for b in range(nb) for h in range(_NH)]
    ss = [jax.lax.dot_general(qb[r, c], kb[r, c], dn,
                              preferred_element_type=jnp.float32)
          for r, c in pairs]
    ps = [jnp.exp2(s - jnp.max(s, axis=-1, keepdims=True))
          .astype(jnp.bfloat16) for s in ss]
    ctx_parts = []
    for (r, c), p in zip(pairs, ps):
        num = jnp.dot(p, vb[r, c], preferred_element_type=jnp.float32)
        den = jnp.dot(p, ones_dh, preferred_element_type=jnp.float32)
        ctx_parts.append((num / den).astype(jnp.bfloat16))
    row_blocks = [
        jnp.concatenate(ctx_parts[b * _NH:(b + 1) * _NH], axis=1)
        for b in range(nb)]
    ctxb = jnp.concatenate(row_blocks, axis=0)

    h_out = jax.lax.dot_general(ctxb, wo_ref[...], dn,
                                preferred_element_type=jnp.float32)
    h_out = h_out + bo_ref[...] + x
    mean = jnp.mean(h_out, axis=-1, keepdims=True)
    c = h_out - mean
    var = jnp.mean(c * c, axis=-1, keepdims=True)
    y = c * jax.lax.rsqrt(var + eps) * g_ref[...] + be_ref[...]
    o_ref[...] = y.astype(o_ref.dtype)


def kernel(hidden_states, wq, wk, wv, wo, bq, bk, bv, bo, gamma, beta):
    B, S, H = hidden_states.shape
    nh = _NH
    dh = H // nh
    M = B * S
    dtype = hidden_states.dtype

    nb = 16
    while B % nb:
        nb -= 1
    tm = nb * S
    grid = (B // nb,)

    x2 = hidden_states.reshape(M, H)
    scale = math.log2(math.e) / math.sqrt(dh)
    wqkv = jnp.concatenate([wq * scale, wk, wv], axis=0).astype(jnp.bfloat16)
    bqkv = jnp.concatenate([bq * scale, bk, bv]).reshape(1, 3 * H)
    bqkv = bqkv.astype(jnp.float32)
    wob = wo.astype(jnp.bfloat16)
    bo2 = bo.reshape(1, H).astype(jnp.float32)
    g2 = gamma.reshape(1, H).astype(jnp.float32)
    be2 = beta.reshape(1, H).astype(jnp.float32)

    row_spec = pl.BlockSpec((tm, H), lambda i: (i, 0))
    wqkv_spec = pl.BlockSpec((3 * H, H), lambda i: (0, 0))
    wt_spec = pl.BlockSpec((H, H), lambda i: (0, 0))
    vecw_spec = pl.BlockSpec((1, 3 * H), lambda i: (0, 0))
    vec_spec = pl.BlockSpec((1, H), lambda i: (0, 0))

    out = pl.pallas_call(
        functools.partial(_fused_kernel, nb=nb, sb=S, dh=dh, eps=1e-12),
        out_shape=jax.ShapeDtypeStruct((M, H), dtype),
        grid=grid,
        in_specs=[row_spec, wqkv_spec, wt_spec,
                  vecw_spec, vec_spec, vec_spec, vec_spec],
        out_specs=row_spec,
        compiler_params=pltpu.CompilerParams(
            dimension_semantics=("parallel",),
            vmem_limit_bytes=60 * 1024 * 1024,
        ),
    )(x2, wqkv, wob, bqkv, bo2, g2, be2)

    return out.reshape(B, S, H)


# speedup vs baseline: 1.0578x; 1.0578x over previous
import functools
import math

import jax
import jax.numpy as jnp
from jax.experimental import pallas as pl
from jax.experimental.pallas import tpu as pltpu

_NH = 12


def _fused_kernel(x_ref, wqkv_ref, wo_ref, bqkv_ref, bo_ref, g_ref,
                  be_ref, o_ref, *, nb, sb, dh, eps):
    x = x_ref[...]
    xb = x.astype(jnp.bfloat16)
    dn = (((1,), (1,)), ((), ()))
    H = x.shape[1]

    qkv = jax.lax.dot_general(xb, wqkv_ref[...], dn,
                              preferred_element_type=jnp.float32)
    qkvb = (qkv + bqkv_ref[...]).astype(jnp.bfloat16)
    qb = qkvb[:, :H]
    kb = qkvb[:, H:2 * H]
    vb = qkvb[:, 2 * H:]

    ones_dh = jnp.ones((sb, dh), dtype=jnp.bfloat16)
    q_h = [qb[:, h * dh:(h + 1) * dh] for h in range(_NH)]
    k_h = [kb[:, h * dh:(h + 1) * dh] for h in range(_NH)]
    pairs = [(slice(b * sb, (b + 1) * sb), slice(h * dh, (h + 1) * dh), h)
             for b in range(nb) for h in range(_NH)]
    ss = [jax.lax.dot_general(q_h[h][r], k_h[h][r], dn,
                              preferred_element_type=jnp.float32)
          for r, c, h in pairs]
    ps = [jnp.exp2(s - jnp.max(s, axis=-1, keepdims=True))
          .astype(jnp.bfloat16) for s in ss]
    ctx_parts = []
    for (r, c, h), p in zip(pairs, ps):
        num = jnp.dot(p, vb[r, c], preferred_element_type=jnp.float32)
        den = jnp.dot(p, ones_dh, preferred_element_type=jnp.float32)
        ctx_parts.append((num / den).astype(jnp.bfloat16))
    row_blocks = [
        jnp.concatenate(ctx_parts[b * _NH:(b + 1) * _NH], axis=1)
        for b in range(nb)]
    ctxb = jnp.concatenate(row_blocks, axis=0)

    h_out = jax.lax.dot_general(ctxb, wo_ref[...], dn,
                                preferred_element_type=jnp.float32)
    h_out = h_out + bo_ref[...] + x
    mean = jnp.mean(h_out, axis=-1, keepdims=True)
    c = h_out - mean
    var = jnp.mean(c * c, axis=-1, keepdims=True)
    y = c * jax.lax.rsqrt(var + eps) * g_ref[...] + be_ref[...]
    o_ref[...] = y.astype(o_ref.dtype)


def kernel(hidden_states, wq, wk, wv, wo, bq, bk, bv, bo, gamma, beta):
    B, S, H = hidden_states.shape
    nh = _NH
    dh = H // nh
    M = B * S
    dtype = hidden_states.dtype

    nb = 8
    while B % nb:
        nb -= 1
    tm = nb * S
    grid = (B // nb,)

    x2 = hidden_states.reshape(M, H)
    scale = math.log2(math.e) / math.sqrt(dh)
    wqkv = jnp.concatenate([wq * scale, wk, wv], axis=0).astype(jnp.bfloat16)
    bqkv = jnp.concatenate([bq * scale, bk, bv]).reshape(1, 3 * H)
    bqkv = bqkv.astype(jnp.float32)
    wob = wo.astype(jnp.bfloat16)
    bo2 = bo.reshape(1, H).astype(jnp.float32)
    g2 = gamma.reshape(1, H).astype(jnp.float32)
    be2 = beta.reshape(1, H).astype(jnp.float32)

    row_spec = pl.BlockSpec((tm, H), lambda i: (i, 0))
    wqkv_spec = pl.BlockSpec((3 * H, H), lambda i: (0, 0))
    wt_spec = pl.BlockSpec((H, H), lambda i: (0, 0))
    vecw_spec = pl.BlockSpec((1, 3 * H), lambda i: (0, 0))
    vec_spec = pl.BlockSpec((1, H), lambda i: (0, 0))

    out = pl.pallas_call(
        functools.partial(_fused_kernel, nb=nb, sb=S, dh=dh, eps=1e-12),
        out_shape=jax.ShapeDtypeStruct((M, H), dtype),
        grid=grid,
        in_specs=[row_spec, wqkv_spec, wt_spec,
                  vecw_spec, vec_spec, vec_spec, vec_spec],
        out_specs=row_spec,
        compiler_params=pltpu.CompilerParams(
            dimension_semantics=("parallel",),
            vmem_limit_bytes=48 * 1024 * 1024,
        ),
    )(x2, wqkv, wob, bqkv, bo2, g2, be2)

    return out.reshape(B, S, H)


# speedup vs baseline: 1.0592x; 1.0014x over previous
import functools
import math

import jax
import jax.numpy as jnp
from jax.experimental import pallas as pl
from jax.experimental.pallas import tpu as pltpu

_NH = 12


def _fused_kernel(x_ref, wqkv_ref, wo_ref, bqkv_ref, bo_ref, g_ref,
                  be_ref, o_ref, *, nb, sb, dh, eps):
    x = x_ref[...]
    xb = x.astype(jnp.bfloat16)
    dn = (((1,), (1,)), ((), ()))
    H = x.shape[1]

    qkv = jax.lax.dot_general(xb, wqkv_ref[...], dn,
                              preferred_element_type=jnp.float32)
    qkvb = (qkv + bqkv_ref[...]).astype(jnp.bfloat16)
    qb = qkvb[:, :H]
    kb = qkvb[:, H:2 * H]
    vb = qkvb[:, 2 * H:]

    ones_dh = jnp.ones((sb, dh), dtype=jnp.bfloat16)
    pairs = [(slice(b * sb, (b + 1) * sb), slice(h * dh, (h + 1) * dh))
             for b in range(nb) for h in range(_NH)]
    ss = [jax.lax.dot_general(qb[r, c], kb[r, c], dn,
                              preferred_element_type=jnp.float32)
          for r, c in pairs]
    ps = [jnp.exp2(s - jnp.max(s, axis=-1, keepdims=True))
          .astype(jnp.bfloat16) for s in ss]
    ctx_parts = []
    for (r, c), p in zip(pairs, ps):
        num = jnp.dot(p, vb[r, c], preferred_element_type=jnp.float32)
        den = jnp.dot(p, ones_dh, preferred_element_type=jnp.float32)
        ctx_parts.append((num / den).astype(jnp.bfloat16))
    row_blocks = [
        jnp.concatenate(ctx_parts[b * _NH:(b + 1) * _NH], axis=1)
        for b in range(nb)]
    ctxb = jnp.concatenate(row_blocks, axis=0)

    h_out = jax.lax.dot_general(ctxb, wo_ref[...], dn,
                                preferred_element_type=jnp.float32)
    h_out = h_out + bo_ref[...] + x
    mean = jnp.mean(h_out, axis=-1, keepdims=True)
    c = h_out - mean
    var = jnp.mean(c * c, axis=-1, keepdims=True)
    y = c * jax.lax.rsqrt(var + eps) * g_ref[...] + be_ref[...]
    o_ref[...] = y.astype(o_ref.dtype)


def kernel(hidden_states, wq, wk, wv, wo, bq, bk, bv, bo, gamma, beta):
    B, S, H = hidden_states.shape
    nh = _NH
    dh = H // nh
    M = B * S
    dtype = hidden_states.dtype

    nb = 8
    while B % nb:
        nb -= 1
    tm = nb * S
    grid = (B // nb,)

    x2 = hidden_states.reshape(M, H)
    scale = math.log2(math.e) / math.sqrt(dh)
    wqkv = jnp.concatenate([wq * scale, wk, wv], axis=0).astype(jnp.bfloat16)
    bqkv = jnp.concatenate([bq * scale, bk, bv]).reshape(1, 3 * H)
    bqkv = bqkv.astype(jnp.float32)
    wob = wo.astype(jnp.bfloat16)
    bo2 = bo.reshape(1, H).astype(jnp.float32)
    g2 = gamma.reshape(1, H).astype(jnp.float32)
    be2 = beta.reshape(1, H).astype(jnp.float32)

    row_spec = pl.BlockSpec((tm, H), lambda i: (i, 0))
    wqkv_spec = pl.BlockSpec((3 * H, H), lambda i: (0, 0))
    wt_spec = pl.BlockSpec((H, H), lambda i: (0, 0))
    vecw_spec = pl.BlockSpec((1, 3 * H), lambda i: (0, 0))
    vec_spec = pl.BlockSpec((1, H), lambda i: (0, 0))

    out = pl.pallas_call(
        functools.partial(_fused_kernel, nb=nb, sb=S, dh=dh, eps=1e-12),
        out_shape=jax.ShapeDtypeStruct((M, H), dtype),
        grid=grid,
        in_specs=[row_spec, wqkv_spec, wt_spec,
                  vecw_spec, vec_spec, vec_spec, vec_spec],
        out_specs=row_spec,
        compiler_params=pltpu.CompilerParams(
            dimension_semantics=("parallel",),
            vmem_limit_bytes=48 * 1024 * 1024,
        ),
    )(x2, wqkv, wob, bqkv, bo2, g2, be2)

    return out.reshape(B, S, H)
